# baseline (speedup 1.0000x reference)
import functools

import jax
import jax.numpy as jnp
from jax import lax
from jax.experimental import pallas as pl
from jax.experimental.pallas import tpu as pltpu
from jax.experimental.pallas import tpu_sc as plsc

L = 16


@functools.lru_cache(maxsize=None)
def _build_sc_gather(n, c, m, nc, ns):
    nw = nc * ns
    rpw = n // nw
    chunk = 16
    nchunk = rpw // chunk
    in_e = chunk * c
    out_e = chunk * m
    nv = out_e // L

    mesh = plsc.VectorSubcoreMesh(core_axis_name="c", subcore_axis_name="s")

    @functools.partial(
        pl.kernel,
        out_type=jax.ShapeDtypeStruct((n * m,), jnp.float32),
        mesh=mesh,
        scratch_types=[
            pltpu.VMEM((out_e,), jnp.int32),
            pltpu.VMEM((in_e,), jnp.float32),
            pltpu.VMEM((in_e,), jnp.float32),
            pltpu.VMEM((out_e,), jnp.float32),
            pltpu.VMEM((out_e,), jnp.float32),
            pltpu.SemaphoreType.DMA,
            pltpu.SemaphoreType.DMA,
            pltpu.SemaphoreType.DMA,
            pltpu.SemaphoreType.DMA,
        ],
        compiler_params=pltpu.CompilerParams(needs_layout_passes=False),
    )
    def sc_gather(xf, idxf, outf, idx_v, in0, in1, ob0, ob1,
                  sin0, sin1, sout0, sout1):
        wid = lax.axis_index("s") * nc + lax.axis_index("c")
        in_base = wid * (rpw * c)
        out_base = wid * (rpw * m)

        in_bufs = (in0, in1)
        ob_bufs = (ob0, ob1)
        in_sems = (sin0, sin1)
        out_sems = (sout0, sout1)

        pltpu.sync_copy(idxf, idx_v)

        def start_in(ci, b):
            pltpu.async_copy(
                xf.at[pl.ds(in_base + ci * in_e, in_e)], in_bufs[b],
                in_sems[b])

        def wait_in(b):
            pltpu.make_async_copy(
                xf.at[pl.ds(in_base, in_e)], in_bufs[b], in_sems[b]).wait()

        def start_out(ci, b):
            pltpu.async_copy(
                ob_bufs[b], outf.at[pl.ds(out_base + ci * out_e, out_e)],
                out_sems[b])

        def wait_out(b):
            pltpu.make_async_copy(
                ob_bufs[b], outf.at[pl.ds(out_base, out_e)],
                out_sems[b]).wait()

        def compute(b):
            ib, ob = in_bufs[b], ob_bufs[b]
            for v in range(nv):
                iv = idx_v[pl.ds(v * L, L)]
                ob[pl.ds(v * L, L)] = plsc.load_gather(ib, [iv])

        start_in(0, 0)

        def g_body(g, carry):
            for s in range(2):
                ci = 2 * g + s
                b = s
                if s == 0:
                    start_in(ci + 1, 1)
                else:
                    @pl.when(ci + 1 < nchunk)
                    def _():
                        start_in(ci + 1, 0)
                wait_in(b)

                @pl.when(g > 0)
                def _():
                    wait_out(b)

                compute(b)
                start_out(ci, b)
            return carry

        lax.fori_loop(0, nchunk // 2, g_body, 0)
        wait_out(0)
        wait_out(1)

    return sc_gather, chunk


def kernel(x, mask):
    n, c = x.shape
    (m,) = mask.shape
    info = plsc.get_sparse_core_info()
    fn, chunk = _build_sc_gather(n, c, m, info.num_cores, info.num_subcores)
    idxp = (jnp.arange(chunk, dtype=jnp.int32)[:, None] * c
            + mask[None, :].astype(jnp.int32)).reshape(-1)
    outf = fn(x.reshape(-1), idxp)
    return outf.reshape(n, m)


# speedup vs baseline: 1.6385x; 1.6385x over previous
import functools

import jax
import jax.numpy as jnp
from jax import lax
from jax.experimental import pallas as pl
from jax.experimental.pallas import tpu as pltpu
from jax.experimental.pallas import tpu_sc as plsc

L = 16


@functools.lru_cache(maxsize=None)
def _build_sc_gather(n, c, m, nc, ns):
    nw = nc * ns
    rpw = n // nw
    chunk = 16
    nchunk = rpw // chunk
    mfull = m // L
    tail = m % L
    nmv = mfull + (1 if tail else 0)

    mesh = plsc.VectorSubcoreMesh(core_axis_name="c", subcore_axis_name="s")

    @functools.partial(
        pl.kernel,
        out_type=jax.ShapeDtypeStruct((n, m), jnp.float32),
        mesh=mesh,
        scratch_types=[
            pltpu.VMEM((nmv * L,), jnp.int32),
            pltpu.VMEM((chunk, c), jnp.float32),
            pltpu.VMEM((chunk, c), jnp.float32),
            pltpu.VMEM((chunk, m), jnp.float32),
            pltpu.VMEM((chunk, m), jnp.float32),
            pltpu.SemaphoreType.DMA,
            pltpu.SemaphoreType.DMA,
            pltpu.SemaphoreType.DMA,
            pltpu.SemaphoreType.DMA,
        ],
        compiler_params=pltpu.CompilerParams(needs_layout_passes=False),
    )
    def sc_gather(xf, idxf, outf, idx_v, in0, in1, ob0, ob1,
                  sin0, sin1, sout0, sout1):
        wid = lax.axis_index("s") * nc + lax.axis_index("c")
        row_base = wid * rpw

        in_bufs = (in0, in1)
        ob_bufs = (ob0, ob1)
        in_sems = (sin0, sin1)
        out_sems = (sout0, sout1)

        pltpu.sync_copy(idxf, idx_v)
        mvecs = [idx_v[pl.ds(k * L, L)] for k in range(nmv)]
        lane = lax.broadcasted_iota(jnp.int32, (L,), 0)
        tailmask = lane < tail
        c_tail = lane + mfull * L

        def start_in(ci, b):
            pltpu.async_copy(
                xf.at[pl.ds(row_base + ci * chunk, chunk)], in_bufs[b],
                in_sems[b])

        def wait_in(b):
            pltpu.make_async_copy(
                xf.at[pl.ds(row_base, chunk)], in_bufs[b], in_sems[b]).wait()

        def start_out(ci, b):
            pltpu.async_copy(
                ob_bufs[b], outf.at[pl.ds(row_base + ci * chunk, chunk)],
                out_sems[b])

        def wait_out(b):
            pltpu.make_async_copy(
                ob_bufs[b], outf.at[pl.ds(row_base, chunk)],
                out_sems[b]).wait()

        def compute(b):
            ib, ob = in_bufs[b], ob_bufs[b]
            for r in range(chunk):
                rsp = jnp.full((L,), r, jnp.int32)
                for k in range(mfull):
                    ob[r, pl.ds(k * L, L)] = plsc.load_gather(
                        ib, [rsp, mvecs[k]])
                if tail:
                    g = plsc.load_gather(ib, [rsp, mvecs[mfull]])
                    plsc.store_scatter(ob, [rsp, c_tail], g, mask=tailmask)

        start_in(0, 0)

        def g_body(g, carry):
            for s in range(2):
                ci = 2 * g + s
                b = s
                if s == 0:
                    start_in(ci + 1, 1)
                else:
                    @pl.when(ci + 1 < nchunk)
                    def _():
                        start_in(ci + 1, 0)
                wait_in(b)

                @pl.when(g > 0)
                def _():
                    wait_out(b)

                compute(b)
                start_out(ci, b)
            return carry

        lax.fori_loop(0, nchunk // 2, g_body, 0)
        wait_out(0)
        wait_out(1)

    return sc_gather, chunk


def kernel(x, mask):
    n, c = x.shape
    (m,) = mask.shape
    info = plsc.get_sparse_core_info()
    fn, chunk = _build_sc_gather(n, c, m, info.num_cores, info.num_subcores)
    (m,) = mask.shape
    pad = (-m) % L
    idxp = jnp.concatenate(
        [mask.astype(jnp.int32), jnp.zeros((pad,), jnp.int32)])
    return fn(x, idxp)


# speedup vs baseline: 1.6460x; 1.0046x over previous
import functools

import jax
import jax.numpy as jnp
from jax import lax
from jax.experimental import pallas as pl
from jax.experimental.pallas import tpu as pltpu
from jax.experimental.pallas import tpu_sc as plsc

L = 16


@functools.lru_cache(maxsize=None)
def _build_sc_gather(n, c, m, nc, ns):
    nw = nc * ns
    rpw = n // nw
    chunk = 16
    nchunk = rpw // chunk
    mfull = m // L
    tail = m % L
    nmv = mfull + (1 if tail else 0)

    mesh = plsc.VectorSubcoreMesh(core_axis_name="c", subcore_axis_name="s")

    @functools.partial(
        pl.kernel,
        out_type=jax.ShapeDtypeStruct((n, m), jnp.float32),
        mesh=mesh,
        scratch_types=[
            pltpu.VMEM((nmv * L,), jnp.int32),
            pltpu.VMEM((chunk, c), jnp.float32),
            pltpu.VMEM((chunk, c), jnp.float32),
            pltpu.VMEM((chunk, m), jnp.float32),
            pltpu.VMEM((chunk, m), jnp.float32),
            pltpu.SemaphoreType.DMA,
            pltpu.SemaphoreType.DMA,
            pltpu.SemaphoreType.DMA,
            pltpu.SemaphoreType.DMA,
        ],
        compiler_params=pltpu.CompilerParams(
            needs_layout_passes=False, use_tc_tiling_on_sc=True),
    )
    def sc_gather(xf, idxf, outf, idx_v, in0, in1, ob0, ob1,
                  sin0, sin1, sout0, sout1):
        wid = lax.axis_index("s") * nc + lax.axis_index("c")
        row_base = wid * rpw

        in_bufs = (in0, in1)
        ob_bufs = (ob0, ob1)
        in_sems = (sin0, sin1)
        out_sems = (sout0, sout1)

        pltpu.sync_copy(idxf, idx_v)
        mvecs = [idx_v[pl.ds(k * L, L)] for k in range(nmv)]
        lane = lax.broadcasted_iota(jnp.int32, (L,), 0)
        tailmask = lane < tail
        c_tail = lane + mfull * L

        def start_in(ci, b):
            pltpu.async_copy(
                xf.at[pl.ds(row_base + ci * chunk, chunk)], in_bufs[b],
                in_sems[b])

        def wait_in(b):
            pltpu.make_async_copy(
                xf.at[pl.ds(row_base, chunk)], in_bufs[b], in_sems[b]).wait()

        def start_out(ci, b):
            pltpu.async_copy(
                ob_bufs[b], outf.at[pl.ds(row_base + ci * chunk, chunk)],
                out_sems[b])

        def wait_out(b):
            pltpu.make_async_copy(
                ob_bufs[b], outf.at[pl.ds(row_base, chunk)],
                out_sems[b]).wait()

        def compute(b):
            ib, ob = in_bufs[b], ob_bufs[b]
            for r in range(chunk):
                rsp = jnp.full((L,), r, jnp.int32)
                for k in range(mfull):
                    ob[r, pl.ds(k * L, L)] = plsc.load_gather(
                        ib, [rsp, mvecs[k]])
                if tail:
                    g = plsc.load_gather(ib, [rsp, mvecs[mfull]])
                    plsc.store_scatter(ob, [rsp, c_tail], g, mask=tailmask)

        start_in(0, 0)

        def g_body(g, carry):
            for s in range(2):
                ci = 2 * g + s
                b = s
                if s == 0:
                    start_in(ci + 1, 1)
                else:
                    @pl.when(ci + 1 < nchunk)
                    def _():
                        start_in(ci + 1, 0)
                wait_in(b)

                @pl.when(g > 0)
                def _():
                    wait_out(b)

                compute(b)
                start_out(ci, b)
            return carry

        lax.fori_loop(0, nchunk // 2, g_body, 0)
        wait_out(0)
        wait_out(1)

    return sc_gather, chunk


def kernel(x, mask):
    n, c = x.shape
    (m,) = mask.shape
    info = plsc.get_sparse_core_info()
    fn, chunk = _build_sc_gather(n, c, m, info.num_cores, info.num_subcores)
    (m,) = mask.shape
    pad = (-m) % L
    idxp = jnp.concatenate(
        [mask.astype(jnp.int32), jnp.zeros((pad,), jnp.int32)])
    return fn(x, idxp)
